# baseline (speedup 1.0000x reference)
import functools

import jax
import jax.numpy as jnp
from jax import lax
from jax.experimental import pallas as pl
from jax.experimental.pallas import tpu as pltpu
from jax.experimental.pallas import tpu_sc as plsc

f32 = jnp.float32
i32 = jnp.int32

_N = 10000
_E = 320000
_IN = 128
_HID = 256
_OUT = 128

_NP = 10240
_RB = 1024
_RPT = _NP // 16
_K = 128
_NCH = 160
_EP = 16 * _NCH * _K
_CH = _EP // _K
_NSC = 2
_F = 128

_mesh = plsc.VectorSubcoreMesh(core_axis_name="c", subcore_axis_name="s")


_BLK = 16
_NBLK = _NCH // _BLK

@functools.partial(
    pl.kernel,
    out_type=jax.ShapeDtypeStruct((_NSC * _NP, _F), f32),
    mesh=_mesh,
    scratch_types=[
        pltpu.VMEM_SHARED((_NP, _F), f32),
        pltpu.VMEM((_BLK, _K), i32),
        pltpu.VMEM((_BLK, _K), i32),
        pltpu.VMEM((_K, _F), f32),
        pltpu.VMEM((_K, _F), f32),
        pltpu.SemaphoreType.DMA,
        pltpu.SemaphoreType.DMA,
        pltpu.SemaphoreType.DMA,
        pltpu.SemaphoreType.DMA,
    ],
)
def _sc_scatter(g_hbm, srcx_hbm, dstx_hbm, zeros_hbm, out_hbm, acc,
                srcblk, dstblk, r0, r1, g0, g1, s0, s1):
    c = lax.axis_index("c")
    s = lax.axis_index("s")
    row0 = s * _RPT
    bufs = (r0, r1)
    gsems = (g0, g1)
    ssems = (s0, s1)

    chunk0 = c * _CH + s * _NCH
    pltpu.sync_copy(zeros_hbm, acc.at[pl.ds(row0, _RPT)])
    plsc.subcore_barrier()

    def issue_gather(t, b):
        pltpu.async_copy(g_hbm.at[srcblk.at[t]], bufs[b], gsems[b])

    def wait_gather(t, b):
        pltpu.make_async_copy(g_hbm.at[srcblk.at[t]], bufs[b], gsems[b]).wait()

    def issue_scatter(t, b):
        pltpu.async_copy(bufs[b], acc.at[dstblk.at[t]], ssems[b], add=True)

    def wait_scatter(t, b):
        pltpu.make_async_copy(bufs[b], acc.at[dstblk.at[t]], ssems[b]).wait()

    def blk_body(iblk, carry):
        pltpu.sync_copy(srcx_hbm.at[pl.ds(chunk0 + iblk * _BLK, _BLK)],
                        srcblk)
        pltpu.sync_copy(dstx_hbm.at[pl.ds(chunk0 + iblk * _BLK, _BLK)],
                        dstblk)
        issue_gather(0, 0)
        for u in range(_BLK):
            b = u % 2
            wait_gather(u, b)
            if u + 1 < _BLK:
                if u >= 1:
                    wait_scatter(u - 1, 1 - b)
                issue_gather(u + 1, 1 - b)
            issue_scatter(u, b)
        wait_scatter(_BLK - 2, 0)
        wait_scatter(_BLK - 1, 1)
        return carry

    lax.fori_loop(0, _NBLK, blk_body, 0)

    plsc.subcore_barrier()
    for t in range(_RPT // _K):
        r = row0 + t * _K
        pltpu.sync_copy(acc.at[pl.ds(r, _K)], r0)
        pltpu.sync_copy(r0, out_hbm.at[pl.ds(c * _NP + r, _K)])


def _tc1_body(x_ref, w_ref, deg_ref, gs_ref, gf_ref, dinv_ref):
    deg = deg_ref[0, :, 0:1] + deg_ref[1, :, 0:1] + 1.0
    dinv = lax.rsqrt(deg)
    h = jnp.dot(x_ref[...], w_ref[...], preferred_element_type=f32)
    g = h * dinv
    half = _HID // 2
    gs_ref[0] = g[:, :half]
    gs_ref[1] = g[:, half:]
    gf_ref[...] = g
    dinv_ref[...] = dinv


def _tc1(x_pad, W1, degacc):
    grid = (_NP // _RB,)
    return pl.pallas_call(
        _tc1_body,
        grid=grid,
        in_specs=[
            pl.BlockSpec((_RB, _IN), lambda i: (i, 0)),
            pl.BlockSpec((_IN, _HID), lambda i: (0, 0)),
            pl.BlockSpec((2, _RB, 128), lambda i: (0, i, 0)),
        ],
        out_specs=[
            pl.BlockSpec((2, _RB, _HID // 2), lambda i: (0, i, 0)),
            pl.BlockSpec((_RB, _HID), lambda i: (i, 0)),
            pl.BlockSpec((_RB, 1), lambda i: (i, 0)),
        ],
        out_shape=[
            jax.ShapeDtypeStruct((2, _NP, _HID // 2), f32),
            jax.ShapeDtypeStruct((_NP, _HID), f32),
            jax.ShapeDtypeStruct((_NP, 1), f32),
        ],
    )(x_pad, W1, degacc)


def _tc2_body(acc_ref, gf_ref, dinv_ref, b_ref, w_ref, g2f_ref):
    accf = jnp.concatenate([acc_ref[0], acc_ref[1]], axis=1)
    dinv = dinv_ref[...]
    h2 = jnp.maximum(dinv * (accf + gf_ref[...]) + b_ref[...], 0.0)
    g2f_ref[...] = jnp.dot(h2, w_ref[...], preferred_element_type=f32) * dinv


def _tc2(acc1, g1f, dinv, b1, W2):
    grid = (_NP // _RB,)
    return pl.pallas_call(
        _tc2_body,
        grid=grid,
        in_specs=[
            pl.BlockSpec((2, _RB, _HID // 2), lambda i: (0, i, 0)),
            pl.BlockSpec((_RB, _HID), lambda i: (i, 0)),
            pl.BlockSpec((_RB, 1), lambda i: (i, 0)),
            pl.BlockSpec((1, _HID), lambda i: (0, 0)),
            pl.BlockSpec((_HID, _OUT), lambda i: (0, 0)),
        ],
        out_specs=pl.BlockSpec((_RB, _OUT), lambda i: (i, 0)),
        out_shape=jax.ShapeDtypeStruct((_NSC * _NP, _OUT), f32),
    )(acc1, g1f, dinv, b1, W2)


def _tc3_body(acc_ref, g2f_ref, dinv_ref, b_ref, out_ref):
    accf = acc_ref[0] + acc_ref[1]
    out_ref[...] = dinv_ref[...] * (accf + g2f_ref[...]) + b_ref[...]


def _tc3(acc2, g2f, dinv, b2):
    grid = (_NP // _RB,)
    return pl.pallas_call(
        _tc3_body,
        grid=grid,
        in_specs=[
            pl.BlockSpec((2, _RB, _OUT), lambda i: (0, i, 0)),
            pl.BlockSpec((_RB, _OUT), lambda i: (i, 0)),
            pl.BlockSpec((_RB, 1), lambda i: (i, 0)),
            pl.BlockSpec((1, _OUT), lambda i: (0, 0)),
        ],
        out_specs=pl.BlockSpec((_RB, _OUT), lambda i: (i, 0)),
        out_shape=jax.ShapeDtypeStruct((_NP, _OUT), f32),
    )(acc2, g2f, dinv, b2)


@jax.jit
def _run(x, edge_index, W1, b1, W2, b2):
    ei = edge_index.astype(i32)
    pad = jnp.full((_EP - _E,), _N, dtype=i32)
    src2d = jnp.concatenate([ei[0], pad]).reshape(_CH, _K)
    dst2d = jnp.concatenate([ei[1], pad]).reshape(_CH, _K)

    x_pad = jnp.zeros((_NP, _IN), f32).at[:_N].set(x)
    zeros_acc = jnp.zeros((_RPT, _F), f32)

    parity = (jnp.arange(_EP, dtype=i32) % 2).reshape(_CH, _K)
    trash = jnp.full((_CH, _K), _N, i32)
    srcx_fs = jnp.concatenate([src2d, src2d + _NP])
    dstx_fs = jnp.concatenate([dst2d, dst2d])
    srcx_eo = jnp.concatenate([src2d, src2d])
    dstx_eo = jnp.concatenate([jnp.where(parity == 0, dst2d, trash),
                               jnp.where(parity == 1, dst2d, trash)])

    onehot_tbl = jnp.zeros((_NSC * _NP, _F), f32).at[:, 0].set(1.0)
    degacc = _sc_scatter(onehot_tbl, srcx_eo, dstx_eo, zeros_acc)
    degacc = degacc.reshape(_NSC, _NP, _F)

    g1s, g1f, dinv = _tc1(x_pad, W1, degacc)

    acc1 = _sc_scatter(g1s.reshape(_NSC * _NP, _F), srcx_fs, dstx_fs,
                       zeros_acc)
    acc1 = acc1.reshape(_NSC, _NP, _F)

    g2pad = _tc2(acc1, g1f, dinv, b1.reshape(1, _HID), W2)

    acc2 = _sc_scatter(g2pad, srcx_eo, dstx_eo, zeros_acc)
    acc2 = acc2.reshape(_NSC, _NP, _OUT)

    out = _tc3(acc2, g2pad, dinv, b2.reshape(1, _OUT))
    return out[:_N]


def kernel(x, edge_index, W1, b1, W2, b2):
    return _run(x, edge_index, W1, b1, W2, b2)


# speedup vs baseline: 1.1431x; 1.1431x over previous
import functools

import jax
import jax.numpy as jnp
from jax import lax
from jax.experimental import pallas as pl
from jax.experimental.pallas import tpu as pltpu
from jax.experimental.pallas import tpu_sc as plsc

f32 = jnp.float32
i32 = jnp.int32

_N = 10000
_E = 320000
_IN = 128
_HID = 256
_OUT = 128

_NP = 10240
_RB = 1024
_RPT = _NP // 16
_K = 128
_NCH = 80
_EH = 16 * _NCH * _K
_E2 = _E // 2
_NSC = 2
_F = 128

_mesh = plsc.VectorSubcoreMesh(core_axis_name="c", subcore_axis_name="s")


@functools.partial(
    pl.kernel,
    out_type=jax.ShapeDtypeStruct((_NSC * _NP, _F), f32),
    mesh=_mesh,
    scratch_types=[
        pltpu.VMEM_SHARED((_NP, _F), f32),
        pltpu.VMEM((_K,), i32),
        pltpu.VMEM((_K,), i32),
        pltpu.VMEM((_K, _F), f32),
        pltpu.VMEM((_K, _F), f32),
        pltpu.SemaphoreType.DMA,
    ],
)
def _sc_scatter(g_hbm, srcx_hbm, dstx_hbm, zeros_hbm, out_hbm, acc,
                srcv, dstv, rowsv, stage, sem):
    c = lax.axis_index("c")
    s = lax.axis_index("s")
    row0 = s * _RPT
    pltpu.sync_copy(zeros_hbm, acc.at[pl.ds(row0, _RPT)])
    plsc.subcore_barrier()

    base = c * _EH + s * (_NCH * _K)

    def body(t, carry):
        off = base + t * _K
        pltpu.sync_copy(srcx_hbm.at[pl.ds(off, _K)], srcv)
        pltpu.sync_copy(dstx_hbm.at[pl.ds(off, _K)], dstv)
        pltpu.async_copy(g_hbm.at[srcv], rowsv, sem).wait()
        pltpu.sync_copy(rowsv, acc.at[dstv], add=True)
        return carry

    lax.fori_loop(0, _NCH, body, 0)
    plsc.subcore_barrier()
    for t in range(_RPT // _K):
        r = row0 + t * _K
        pltpu.sync_copy(acc.at[pl.ds(r, _K)], stage)
        pltpu.sync_copy(stage, out_hbm.at[pl.ds(c * _NP + r, _K)])


def _tc1_body(x_ref, w_ref, deg_ref, gs_ref, gf_ref, dinv_ref):
    deg = deg_ref[0, :, 0:1] + deg_ref[1, :, 0:1] + 1.0
    dinv = lax.rsqrt(deg)
    h = jnp.dot(x_ref[...], w_ref[...], preferred_element_type=f32)
    g = h * dinv
    half = _HID // 2
    gs_ref[0] = g[:, :half]
    gs_ref[1] = g[:, half:]
    gf_ref[...] = g
    dinv_ref[...] = dinv


def _tc1(x_pad, W1, degacc):
    grid = (_NP // _RB,)
    return pl.pallas_call(
        _tc1_body,
        grid=grid,
        in_specs=[
            pl.BlockSpec((_RB, _IN), lambda i: (i, 0)),
            pl.BlockSpec((_IN, _HID), lambda i: (0, 0)),
            pl.BlockSpec((2, _RB, 128), lambda i: (0, i, 0)),
        ],
        out_specs=[
            pl.BlockSpec((2, _RB, _HID // 2), lambda i: (0, i, 0)),
            pl.BlockSpec((_RB, _HID), lambda i: (i, 0)),
            pl.BlockSpec((_RB, 1), lambda i: (i, 0)),
        ],
        out_shape=[
            jax.ShapeDtypeStruct((2, _NP, _HID // 2), f32),
            jax.ShapeDtypeStruct((_NP, _HID), f32),
            jax.ShapeDtypeStruct((_NP, 1), f32),
        ],
    )(x_pad, W1, degacc)


def _tc2_body(acca_ref, accb_ref, gf_ref, dinv_ref, b_ref, w_ref, g2f_ref):
    accf = jnp.concatenate([acca_ref[0] + accb_ref[0],
                            acca_ref[1] + accb_ref[1]], axis=1)
    dinv = dinv_ref[...]
    h2 = jnp.maximum(dinv * (accf + gf_ref[...]) + b_ref[...], 0.0)
    g2f_ref[...] = jnp.dot(h2, w_ref[...], preferred_element_type=f32) * dinv


def _tc2(acc1a, acc1b, g1f, dinv, b1, W2):
    grid = (_NP // _RB,)
    return pl.pallas_call(
        _tc2_body,
        grid=grid,
        in_specs=[
            pl.BlockSpec((2, _RB, _HID // 2), lambda i: (0, i, 0)),
            pl.BlockSpec((2, _RB, _HID // 2), lambda i: (0, i, 0)),
            pl.BlockSpec((_RB, _HID), lambda i: (i, 0)),
            pl.BlockSpec((_RB, 1), lambda i: (i, 0)),
            pl.BlockSpec((1, _HID), lambda i: (0, 0)),
            pl.BlockSpec((_HID, _OUT), lambda i: (0, 0)),
        ],
        out_specs=pl.BlockSpec((_RB, _OUT), lambda i: (i, 0)),
        out_shape=jax.ShapeDtypeStruct((_NSC * _NP, _OUT), f32),
    )(acc1a, acc1b, g1f, dinv, b1, W2)


def _tc3_body(acc_ref, g2f_ref, dinv_ref, b_ref, out_ref):
    accf = acc_ref[0] + acc_ref[1]
    out_ref[...] = dinv_ref[...] * (accf + g2f_ref[...]) + b_ref[...]


def _tc3(acc2, g2f, dinv, b2):
    grid = (_NP // _RB,)
    return pl.pallas_call(
        _tc3_body,
        grid=grid,
        in_specs=[
            pl.BlockSpec((2, _RB, _OUT), lambda i: (0, i, 0)),
            pl.BlockSpec((_RB, _OUT), lambda i: (i, 0)),
            pl.BlockSpec((_RB, 1), lambda i: (i, 0)),
            pl.BlockSpec((1, _OUT), lambda i: (0, 0)),
        ],
        out_specs=pl.BlockSpec((_RB, _OUT), lambda i: (i, 0)),
        out_shape=jax.ShapeDtypeStruct((_NP, _OUT), f32),
    )(acc2, g2f, dinv, b2)


@jax.jit
def _run(x, edge_index, W1, b1, W2, b2):
    ei = edge_index.astype(i32)
    dum = jnp.full((_EH - _E2,), _N, dtype=i32)
    srcA, srcB = ei[0, :_E2], ei[0, _E2:]
    dstA, dstB = ei[1, :_E2], ei[1, _E2:]

    x_pad = jnp.zeros((_NP, _IN), f32).at[:_N].set(x)
    zeros_acc = jnp.zeros((_RPT, _F), f32)

    srcx_eo = jnp.concatenate([srcA, dum, srcB, dum])
    dstx_eo = jnp.concatenate([dstA, dum, dstB, dum])
    srcx_a = jnp.concatenate([srcA, dum, srcA + _NP, dum])
    dstx_a = jnp.concatenate([dstA, dum, dstA, dum])
    srcx_b = jnp.concatenate([srcB, dum, srcB + _NP, dum])
    dstx_b = jnp.concatenate([dstB, dum, dstB, dum])

    onehot_tbl = jnp.zeros((_NSC * _NP, _F), f32).at[:, 0].set(1.0)
    degacc = _sc_scatter(onehot_tbl, srcx_eo, dstx_eo, zeros_acc)
    degacc = degacc.reshape(_NSC, _NP, _F)

    g1s, g1f, dinv = _tc1(x_pad, W1, degacc)

    g1tbl = g1s.reshape(_NSC * _NP, _F)
    acc1a = _sc_scatter(g1tbl, srcx_a, dstx_a, zeros_acc).reshape(
        _NSC, _NP, _F)
    acc1b = _sc_scatter(g1tbl, srcx_b, dstx_b, zeros_acc).reshape(
        _NSC, _NP, _F)

    g2pad = _tc2(acc1a, acc1b, g1f, dinv, b1.reshape(1, _HID), W2)

    acc2 = _sc_scatter(g2pad, srcx_eo, dstx_eo, zeros_acc)
    acc2 = acc2.reshape(_NSC, _NP, _OUT)

    out = _tc3(acc2, g2pad, dinv, b2.reshape(1, _OUT))
    return out[:_N]


def kernel(x, edge_index, W1, b1, W2, b2):
    return _run(x, edge_index, W1, b1, W2, b2)
